# baseline (speedup 1.0000x reference)
import functools

import jax
import jax.numpy as jnp
from jax import lax
from jax.experimental import pallas as pl
from jax.experimental.pallas import tpu as pltpu
from jax.experimental.pallas import tpu_sc as plsc

NC = 2
NS = 16
LANES = 16
NW = NC * NS


def _make_spmm(V, F, nchunk, C):
    rpt = (V // NS) // 8 * 8
    rem = V - rpt * NS
    nz = rpt // C
    zt = rpt - nz * C
    mesh = plsc.VectorSubcoreMesh(
        core_axis_name="c", subcore_axis_name="s", num_cores=NC,
        num_subcores=NS)

    @functools.partial(
        pl.kernel,
        out_type=jax.ShapeDtypeStruct((NC, V, F), jnp.float32),
        mesh=mesh,
        scratch_types=[
            pltpu.VMEM((nchunk, C), jnp.int32),
            pltpu.VMEM((nchunk, C), jnp.int32),
            pltpu.VMEM((nchunk, C), jnp.float32),
            pltpu.VMEM((2, C, F), jnp.float32),
            pltpu.SemaphoreType.DMA,
            pltpu.SemaphoreType.DMA,
            pltpu.VMEM_SHARED((V, F), jnp.float32),
        ],
    )
    def spmm(x_hbm, col_hbm, row_hbm, w_hbm, out_hbm,
             col_v, row_v, w_v, rows_v, sg0, sg1, acc):
        cid = lax.axis_index("c")
        sid = lax.axis_index("s")
        wid = sid * NC + cid

        pltpu.sync_copy(col_hbm.at[wid], col_v)
        pltpu.sync_copy(row_hbm.at[wid], row_v)
        pltpu.sync_copy(w_hbm.at[wid], w_v)

        sgs = (sg0, sg1)

        def start(j, b):
            pltpu.async_copy(x_hbm.at[col_v.at[j]], rows_v.at[b], sgs[b])

        def finish(j, b):
            pltpu.make_async_copy(x_hbm.at[col_v.at[j]], rows_v.at[b],
                                  sgs[b]).wait()

        def scale_scatter(j, b):
            @plsc.parallel_loop(0, C // LANES, unroll=2)
            def sgroup(g):
                wvec = w_v[j, pl.ds(pl.multiple_of(g * LANES, LANES), LANES)]
                for k in range(LANES):
                    wb = lax.gather(
                        wvec, jnp.full((LANES, 1), k, jnp.int32),
                        lax.GatherDimensionNumbers(
                            offset_dims=(), collapsed_slice_dims=(0,),
                            start_index_map=(0,)),
                        (1,), mode=lax.GatherScatterMode.PROMISE_IN_BOUNDS)
                    e = g * LANES + k
                    for t in range(F // LANES):
                        rows_v[b, e, pl.ds(t * LANES, LANES)] = (
                            rows_v[b, e, pl.ds(t * LANES, LANES)] * wb)

            pltpu.sync_copy(rows_v.at[b], acc.at[row_v.at[j]], add=True)

        start(0, 0)

        zero = jnp.zeros((LANES,), jnp.float32)

        def zrow(r, carry):
            for t in range(F // LANES):
                rows_v[1, r, pl.ds(t * LANES, LANES)] = zero
            return carry

        lax.fori_loop(0, C, zrow, 0)
        base_row = sid * rpt
        for z in range(nz):
            pltpu.sync_copy(rows_v.at[1], acc.at[pl.ds(base_row + z * C, C)])
        pltpu.sync_copy(rows_v.at[1].at[pl.ds(0, zt)],
                        acc.at[pl.ds(base_row + nz * C, zt)])

        @pl.when(sid == NS - 1)
        def _zero_tail():
            pltpu.sync_copy(rows_v.at[1].at[pl.ds(0, rem)],
                            acc.at[pl.ds(NS * rpt, rem)])

        plsc.subcore_barrier()

        def pair(jj, carry):
            j0 = 2 * jj
            start(j0 + 1, 1)
            finish(j0, 0)
            scale_scatter(j0, 0)

            @pl.when(jj < nchunk // 2 - 1)
            def _prefetch_even():
                start(j0 + 2, 0)

            finish(j0 + 1, 1)
            scale_scatter(j0 + 1, 1)
            return carry

        lax.fori_loop(0, nchunk // 2, pair, 0)
        plsc.subcore_barrier()

        for z in range(nz):
            pltpu.sync_copy(acc.at[pl.ds(base_row + z * C, C)],
                            out_hbm.at[cid, pl.ds(base_row + z * C, C)])
        pltpu.sync_copy(acc.at[pl.ds(base_row + nz * C, zt)],
                        out_hbm.at[cid, pl.ds(base_row + nz * C, zt)])

        @pl.when(sid == NS - 1)
        def _write_tail():
            pltpu.sync_copy(acc.at[pl.ds(NS * rpt, rem)],
                            out_hbm.at[cid, pl.ds(NS * rpt, rem)])

    return spmm


def _combine_add(V, F, br):
    def body(a_ref, b_ref, o_ref):
        o_ref[...] = a_ref[...] + b_ref[...]

    spec = pl.BlockSpec((br, F), lambda i: (i, 0))
    return pl.pallas_call(
        body,
        grid=(V // br,),
        in_specs=[spec, spec],
        out_specs=spec,
        out_shape=jax.ShapeDtypeStruct((V, F), jnp.float32),
    )


def _combine_cheb(V, F, br):
    def body(a_ref, b_ref, c_ref, o_ref):
        o_ref[...] = 2.0 * (a_ref[...] + b_ref[...]) - c_ref[...]

    spec = pl.BlockSpec((br, F), lambda i: (i, 0))
    return pl.pallas_call(
        body,
        grid=(V // br,),
        in_specs=[spec, spec, spec],
        out_specs=spec,
        out_shape=jax.ShapeDtypeStruct((V, F), jnp.float32),
    )


def _make_project(V, F, Fout, br):
    xspec = pl.BlockSpec((br, F), lambda i: (i, 0))

    def body(x0_ref, x1_ref, x2_ref, p0_ref, p1_ref, w_ref, b_ref, o_ref):
        x1b = x1_ref[...]
        x3b = 2.0 * (p0_ref[...] + p1_ref[...]) - x1b
        acc = jnp.dot(x0_ref[...], w_ref[0],
                      preferred_element_type=jnp.float32,
                      precision=lax.Precision.HIGHEST)
        acc += jnp.dot(x1b, w_ref[1], preferred_element_type=jnp.float32,
                       precision=lax.Precision.HIGHEST)
        acc += jnp.dot(x2_ref[...], w_ref[2],
                       preferred_element_type=jnp.float32,
                       precision=lax.Precision.HIGHEST)
        acc += jnp.dot(x3b, w_ref[3], preferred_element_type=jnp.float32,
                       precision=lax.Precision.HIGHEST)
        o_ref[...] = acc + b_ref[...]

    return pl.pallas_call(
        body,
        grid=(V // br,),
        in_specs=[xspec, xspec, xspec, xspec, xspec,
                  pl.BlockSpec((4, F, Fout), lambda i: (0, 0, 0)),
                  pl.BlockSpec((1, Fout), lambda i: (0, 0))],
        out_specs=pl.BlockSpec((br, Fout), lambda i: (i, 0)),
        out_shape=jax.ShapeDtypeStruct((V, Fout), jnp.float32),
    )


def kernel(inputs, edge_index, edge_weight, weight, bias):
    B, Fin, V, X, Y, Z = inputs.shape
    K, _, Fout = weight.shape
    E = edge_weight.shape[0]
    F = Fin * B * X * Y * Z

    x0 = jnp.transpose(inputs, (2, 1, 0, 3, 4, 5)).reshape(V, F)

    C = 128
    per_w = -(-E // NW)
    nchunk = -(-per_w // C)
    epad = NW * nchunk * C - E
    row = jnp.pad(edge_index[0], (0, epad)).reshape(NW, nchunk, C)
    col = jnp.pad(edge_index[1], (0, epad)).reshape(NW, nchunk, C)
    w = jnp.pad(edge_weight, (0, epad)).reshape(NW, nchunk, C)

    spmm = _make_spmm(V, F, nchunk, C)
    br = 1000
    comb_add = _combine_add(V, F, br)
    comb_cheb = _combine_cheb(V, F, br)
    project = _make_project(V, F, Fout, br)

    p1 = spmm(x0, col, row, w)
    x1 = comb_add(p1[0], p1[1])
    p2 = spmm(x1, col, row, w)
    x2 = comb_cheb(p2[0], p2[1], x0)
    p3 = spmm(x2, col, row, w)
    out = project(x0, x1, x2, p3[0], p3[1], weight, bias.reshape(1, Fout))
    return out.T[None, :, :, None, None, None]


# speedup vs baseline: 1.1765x; 1.1765x over previous
import functools

import jax
import jax.numpy as jnp
from jax import lax
from jax.experimental import pallas as pl
from jax.experimental.pallas import tpu as pltpu
from jax.experimental.pallas import tpu_sc as plsc

NC = 2
NS = 16
LANES = 16
M0 = 64


def _make_spmm(V, F, npair, C):
    rpt = (V // NS) // 8 * 8
    rem = V - rpt * NS
    nz = rpt // C
    zt = rpt - nz * C
    m1 = npair - M0
    mesh = plsc.VectorSubcoreMesh(
        core_axis_name="c", subcore_axis_name="s", num_cores=NC,
        num_subcores=NS)

    @functools.partial(
        pl.kernel,
        out_type=jax.ShapeDtypeStruct((NC, V, F), jnp.float32),
        mesh=mesh,
        scratch_types=[
            pltpu.VMEM((M0, C), jnp.int32),
            pltpu.VMEM((2, C), jnp.int32),
            pltpu.VMEM((2, C), jnp.float32),
            pltpu.VMEM((2, C, F), jnp.float32),
            pltpu.SemaphoreType.DMA,
            pltpu.SemaphoreType.DMA,
            pltpu.SemaphoreType.DMA,
            pltpu.SemaphoreType.DMA,
            pltpu.VMEM_SHARED((V, F), jnp.float32),
        ],
    )
    def spmm(x_hbm, col_hbm, row_hbm, w_hbm, out_hbm,
             col_v, row_v, w_v, rows_v, sg0, sg1, sr0, sr1, acc):
        cid = lax.axis_index("c")
        sid = lax.axis_index("s")
        base_j = jnp.where(cid == 0, 0, M0)
        n_my = jnp.where(cid == 0, M0, m1)

        @pl.when(cid == 0)
        def _stage_col0():
            pltpu.sync_copy(col_hbm.at[sid, pl.ds(0, M0)], col_v)

        @pl.when(cid == 1)
        def _stage_col1():
            pltpu.sync_copy(col_hbm.at[sid, pl.ds(M0, m1)],
                            col_v.at[pl.ds(0, m1)])

        sgs = (sg0, sg1)
        srs = (sr0, sr1)

        def start(j, b):
            pltpu.async_copy(row_hbm.at[sid, base_j + j], row_v.at[b], srs[b])
            pltpu.async_copy(w_hbm.at[sid, base_j + j], w_v.at[b], srs[b])
            pltpu.async_copy(x_hbm.at[col_v.at[j]], rows_v.at[b], sgs[b])

        def finish(j, b):
            pltpu.make_async_copy(row_hbm.at[sid, base_j + j], row_v.at[b],
                                  srs[b]).wait()
            pltpu.make_async_copy(w_hbm.at[sid, base_j + j], w_v.at[b],
                                  srs[b]).wait()
            pltpu.make_async_copy(x_hbm.at[col_v.at[j]], rows_v.at[b],
                                  sgs[b]).wait()

        def scale_scatter(j, b):
            @plsc.parallel_loop(0, C // LANES, unroll=2)
            def sgroup(g):
                wvec = w_v[b, pl.ds(pl.multiple_of(g * LANES, LANES), LANES)]
                for k in range(LANES):
                    wb = lax.gather(
                        wvec, jnp.full((LANES, 1), k, jnp.int32),
                        lax.GatherDimensionNumbers(
                            offset_dims=(), collapsed_slice_dims=(0,),
                            start_index_map=(0,)),
                        (1,), mode=lax.GatherScatterMode.PROMISE_IN_BOUNDS)
                    e = g * LANES + k
                    for t in range(F // LANES):
                        rows_v[b, e, pl.ds(t * LANES, LANES)] = (
                            rows_v[b, e, pl.ds(t * LANES, LANES)] * wb)

            pltpu.sync_copy(rows_v.at[b], acc.at[row_v.at[b]], add=True)

        start(0, 0)

        zero = jnp.zeros((LANES,), jnp.float32)

        def zrow(r, carry):
            for t in range(F // LANES):
                rows_v[1, r, pl.ds(t * LANES, LANES)] = zero
            return carry

        lax.fori_loop(0, C, zrow, 0)
        base_row = sid * rpt
        for z in range(nz):
            pltpu.sync_copy(rows_v.at[1], acc.at[pl.ds(base_row + z * C, C)])
        pltpu.sync_copy(rows_v.at[1].at[pl.ds(0, zt)],
                        acc.at[pl.ds(base_row + nz * C, zt)])

        @pl.when(sid == NS - 1)
        def _zero_tail():
            pltpu.sync_copy(rows_v.at[1].at[pl.ds(0, rem)],
                            acc.at[pl.ds(NS * rpt, rem)])

        plsc.subcore_barrier()

        def pair(jj, carry):
            j0 = 2 * jj
            start(j0 + 1, 1)
            finish(j0, 0)
            scale_scatter(j0, 0)

            @pl.when(j0 + 2 < n_my)
            def _prefetch_even():
                start(j0 + 2, 0)

            finish(j0 + 1, 1)
            scale_scatter(j0 + 1, 1)
            return carry

        lax.fori_loop(0, n_my // 2, pair, 0)
        plsc.subcore_barrier()

        for z in range(nz):
            pltpu.sync_copy(acc.at[pl.ds(base_row + z * C, C)],
                            out_hbm.at[cid, pl.ds(base_row + z * C, C)])
        pltpu.sync_copy(acc.at[pl.ds(base_row + nz * C, zt)],
                        out_hbm.at[cid, pl.ds(base_row + nz * C, zt)])

        @pl.when(sid == NS - 1)
        def _write_tail():
            pltpu.sync_copy(acc.at[pl.ds(NS * rpt, rem)],
                            out_hbm.at[cid, pl.ds(NS * rpt, rem)])

    return spmm


def _combine_add(V, F, br):
    def body(a_ref, b_ref, o_ref):
        o_ref[...] = a_ref[...] + b_ref[...]

    spec = pl.BlockSpec((br, F), lambda i: (i, 0))
    return pl.pallas_call(
        body,
        grid=(V // br,),
        in_specs=[spec, spec],
        out_specs=spec,
        out_shape=jax.ShapeDtypeStruct((V, F), jnp.float32),
    )


def _combine_cheb(V, F, br):
    def body(a_ref, b_ref, c_ref, o_ref):
        o_ref[...] = 2.0 * (a_ref[...] + b_ref[...]) - c_ref[...]

    spec = pl.BlockSpec((br, F), lambda i: (i, 0))
    return pl.pallas_call(
        body,
        grid=(V // br,),
        in_specs=[spec, spec, spec],
        out_specs=spec,
        out_shape=jax.ShapeDtypeStruct((V, F), jnp.float32),
    )


def _make_project(V, F, Fout, br):
    xspec = pl.BlockSpec((br, F), lambda i: (i, 0))

    def body(x0_ref, x1_ref, x2_ref, p0_ref, p1_ref, w_ref, b_ref, o_ref):
        x1b = x1_ref[...]
        x3b = 2.0 * (p0_ref[...] + p1_ref[...]) - x1b
        acc = jnp.dot(x0_ref[...], w_ref[0],
                      preferred_element_type=jnp.float32,
                      precision=lax.Precision.HIGHEST)
        acc += jnp.dot(x1b, w_ref[1], preferred_element_type=jnp.float32,
                       precision=lax.Precision.HIGHEST)
        acc += jnp.dot(x2_ref[...], w_ref[2],
                       preferred_element_type=jnp.float32,
                       precision=lax.Precision.HIGHEST)
        acc += jnp.dot(x3b, w_ref[3], preferred_element_type=jnp.float32,
                       precision=lax.Precision.HIGHEST)
        o_ref[...] = acc + b_ref[...]

    return pl.pallas_call(
        body,
        grid=(V // br,),
        in_specs=[xspec, xspec, xspec, xspec, xspec,
                  pl.BlockSpec((4, F, Fout), lambda i: (0, 0, 0)),
                  pl.BlockSpec((1, Fout), lambda i: (0, 0))],
        out_specs=pl.BlockSpec((br, Fout), lambda i: (i, 0)),
        out_shape=jax.ShapeDtypeStruct((V, Fout), jnp.float32),
    )


def kernel(inputs, edge_index, edge_weight, weight, bias):
    B, Fin, V, X, Y, Z = inputs.shape
    K, _, Fout = weight.shape
    E = edge_weight.shape[0]
    F = Fin * B * X * Y * Z

    x0 = jnp.transpose(inputs, (2, 1, 0, 3, 4, 5)).reshape(V, F)

    C = 128
    per_s = -(-E // NS)
    npair = -(-per_s // C)
    npair += npair % 2
    epad = NS * npair * C - E
    row = jnp.pad(edge_index[0], (0, epad)).reshape(NS, npair, C)
    col = jnp.pad(edge_index[1], (0, epad)).reshape(NS, npair, C)
    w = jnp.pad(edge_weight, (0, epad)).reshape(NS, npair, C)

    spmm = _make_spmm(V, F, npair, C)
    br = 1000
    comb_add = _combine_add(V, F, br)
    comb_cheb = _combine_cheb(V, F, br)
    project = _make_project(V, F, Fout, br)

    p1 = spmm(x0, col, row, w)
    x1 = comb_add(p1[0], p1[1])
    p2 = spmm(x1, col, row, w)
    x2 = comb_cheb(p2[0], p2[1], x0)
    p3 = spmm(x2, col, row, w)
    out = project(x0, x1, x2, p3[0], p3[1], weight, bias.reshape(1, Fout))
    return out.T[None, :, :, None, None, None]


# speedup vs baseline: 1.1778x; 1.0011x over previous
import functools

import jax
import jax.numpy as jnp
from jax import lax
from jax.experimental import pallas as pl
from jax.experimental.pallas import tpu as pltpu
from jax.experimental.pallas import tpu_sc as plsc

NC = 2
NS = 16
LANES = 16
M0 = 56


def _make_spmm(V, F, npair, C):
    rpt = (V // NS) // 8 * 8
    rem = V - rpt * NS
    nz = rpt // C
    zt = rpt - nz * C
    m1 = npair - M0
    mesh = plsc.VectorSubcoreMesh(
        core_axis_name="c", subcore_axis_name="s", num_cores=NC,
        num_subcores=NS)

    @functools.partial(
        pl.kernel,
        out_type=jax.ShapeDtypeStruct((NC, V, F), jnp.float32),
        mesh=mesh,
        scratch_types=[
            pltpu.VMEM((M0, C), jnp.int32),
            pltpu.VMEM((2, C), jnp.int32),
            pltpu.VMEM((2, C), jnp.float32),
            pltpu.VMEM((2, C, F), jnp.float32),
            pltpu.SemaphoreType.DMA,
            pltpu.SemaphoreType.DMA,
            pltpu.SemaphoreType.DMA,
            pltpu.SemaphoreType.DMA,
            pltpu.VMEM_SHARED((V, F), jnp.float32),
        ],
    )
    def spmm(x_hbm, col_hbm, row_hbm, w_hbm, out_hbm,
             col_v, row_v, w_v, rows_v, sg0, sg1, sr0, sr1, acc):
        cid = lax.axis_index("c")
        sid = lax.axis_index("s")
        base_j = jnp.where(cid == 0, 0, M0)
        n_my = jnp.where(cid == 0, M0, m1)

        @pl.when(cid == 0)
        def _stage_col0():
            pltpu.sync_copy(col_hbm.at[sid, pl.ds(0, M0)], col_v)

        @pl.when(cid == 1)
        def _stage_col1():
            pltpu.sync_copy(col_hbm.at[sid, pl.ds(M0, m1)],
                            col_v.at[pl.ds(0, m1)])

        sgs = (sg0, sg1)
        srs = (sr0, sr1)

        def start(j, b):
            pltpu.async_copy(row_hbm.at[sid, base_j + j], row_v.at[b], srs[b])
            pltpu.async_copy(w_hbm.at[sid, base_j + j], w_v.at[b], srs[b])
            pltpu.async_copy(x_hbm.at[col_v.at[j]], rows_v.at[b], sgs[b])

        def finish(j, b):
            pltpu.make_async_copy(row_hbm.at[sid, base_j + j], row_v.at[b],
                                  srs[b]).wait()
            pltpu.make_async_copy(w_hbm.at[sid, base_j + j], w_v.at[b],
                                  srs[b]).wait()
            pltpu.make_async_copy(x_hbm.at[col_v.at[j]], rows_v.at[b],
                                  sgs[b]).wait()

        def scale_scatter(j, b):
            @plsc.parallel_loop(0, C // LANES, unroll=4)
            def sgroup(g):
                wvec = w_v[b, pl.ds(pl.multiple_of(g * LANES, LANES), LANES)]
                for k in range(LANES):
                    wb = lax.gather(
                        wvec, jnp.full((LANES, 1), k, jnp.int32),
                        lax.GatherDimensionNumbers(
                            offset_dims=(), collapsed_slice_dims=(0,),
                            start_index_map=(0,)),
                        (1,), mode=lax.GatherScatterMode.PROMISE_IN_BOUNDS)
                    e = g * LANES + k
                    for t in range(F // LANES):
                        rows_v[b, e, pl.ds(t * LANES, LANES)] = (
                            rows_v[b, e, pl.ds(t * LANES, LANES)] * wb)

            pltpu.sync_copy(rows_v.at[b], acc.at[row_v.at[b]], add=True)

        start(0, 0)

        zero = jnp.zeros((LANES,), jnp.float32)

        def zrow(r, carry):
            for t in range(F // LANES):
                rows_v[1, r, pl.ds(t * LANES, LANES)] = zero
            return carry

        lax.fori_loop(0, C, zrow, 0)
        base_row = sid * rpt
        for z in range(nz):
            pltpu.sync_copy(rows_v.at[1], acc.at[pl.ds(base_row + z * C, C)])
        pltpu.sync_copy(rows_v.at[1].at[pl.ds(0, zt)],
                        acc.at[pl.ds(base_row + nz * C, zt)])

        @pl.when(sid == NS - 1)
        def _zero_tail():
            pltpu.sync_copy(rows_v.at[1].at[pl.ds(0, rem)],
                            acc.at[pl.ds(NS * rpt, rem)])

        plsc.subcore_barrier()

        def pair(jj, carry):
            j0 = 2 * jj
            start(j0 + 1, 1)
            finish(j0, 0)
            scale_scatter(j0, 0)

            @pl.when(j0 + 2 < n_my)
            def _prefetch_even():
                start(j0 + 2, 0)

            finish(j0 + 1, 1)
            scale_scatter(j0 + 1, 1)
            return carry

        lax.fori_loop(0, n_my // 2, pair, 0)
        plsc.subcore_barrier()

        for z in range(nz):
            pltpu.sync_copy(acc.at[pl.ds(base_row + z * C, C)],
                            out_hbm.at[cid, pl.ds(base_row + z * C, C)])
        pltpu.sync_copy(acc.at[pl.ds(base_row + nz * C, zt)],
                        out_hbm.at[cid, pl.ds(base_row + nz * C, zt)])

        @pl.when(sid == NS - 1)
        def _write_tail():
            pltpu.sync_copy(acc.at[pl.ds(NS * rpt, rem)],
                            out_hbm.at[cid, pl.ds(NS * rpt, rem)])

    return spmm


def _combine_add(V, F, br):
    def body(a_ref, b_ref, o_ref):
        o_ref[...] = a_ref[...] + b_ref[...]

    spec = pl.BlockSpec((br, F), lambda i: (i, 0))
    return pl.pallas_call(
        body,
        grid=(V // br,),
        in_specs=[spec, spec],
        out_specs=spec,
        out_shape=jax.ShapeDtypeStruct((V, F), jnp.float32),
    )


def _combine_cheb(V, F, br):
    def body(a_ref, b_ref, c_ref, o_ref):
        o_ref[...] = 2.0 * (a_ref[...] + b_ref[...]) - c_ref[...]

    spec = pl.BlockSpec((br, F), lambda i: (i, 0))
    return pl.pallas_call(
        body,
        grid=(V // br,),
        in_specs=[spec, spec, spec],
        out_specs=spec,
        out_shape=jax.ShapeDtypeStruct((V, F), jnp.float32),
    )


def _make_project(V, F, Fout, br):
    xspec = pl.BlockSpec((br, F), lambda i: (i, 0))

    def body(x0_ref, x1_ref, x2_ref, p0_ref, p1_ref, w_ref, b_ref, o_ref):
        x1b = x1_ref[...]
        x3b = 2.0 * (p0_ref[...] + p1_ref[...]) - x1b
        acc = jnp.dot(x0_ref[...], w_ref[0],
                      preferred_element_type=jnp.float32,
                      precision=lax.Precision.HIGHEST)
        acc += jnp.dot(x1b, w_ref[1], preferred_element_type=jnp.float32,
                       precision=lax.Precision.HIGHEST)
        acc += jnp.dot(x2_ref[...], w_ref[2],
                       preferred_element_type=jnp.float32,
                       precision=lax.Precision.HIGHEST)
        acc += jnp.dot(x3b, w_ref[3], preferred_element_type=jnp.float32,
                       precision=lax.Precision.HIGHEST)
        o_ref[...] = acc + b_ref[...]

    return pl.pallas_call(
        body,
        grid=(V // br,),
        in_specs=[xspec, xspec, xspec, xspec, xspec,
                  pl.BlockSpec((4, F, Fout), lambda i: (0, 0, 0)),
                  pl.BlockSpec((1, Fout), lambda i: (0, 0))],
        out_specs=pl.BlockSpec((br, Fout), lambda i: (i, 0)),
        out_shape=jax.ShapeDtypeStruct((V, Fout), jnp.float32),
    )


def kernel(inputs, edge_index, edge_weight, weight, bias):
    B, Fin, V, X, Y, Z = inputs.shape
    K, _, Fout = weight.shape
    E = edge_weight.shape[0]
    F = Fin * B * X * Y * Z

    x0 = jnp.transpose(inputs, (2, 1, 0, 3, 4, 5)).reshape(V, F)

    C = 128
    per_s = -(-E // NS)
    npair = -(-per_s // C)
    npair += npair % 2
    epad = NS * npair * C - E
    row = jnp.pad(edge_index[0], (0, epad)).reshape(NS, npair, C)
    col = jnp.pad(edge_index[1], (0, epad)).reshape(NS, npair, C)
    w = jnp.pad(edge_weight, (0, epad)).reshape(NS, npair, C)

    spmm = _make_spmm(V, F, npair, C)
    br = 1000
    comb_add = _combine_add(V, F, br)
    comb_cheb = _combine_cheb(V, F, br)
    project = _make_project(V, F, Fout, br)

    p1 = spmm(x0, col, row, w)
    x1 = comb_add(p1[0], p1[1])
    p2 = spmm(x1, col, row, w)
    x2 = comb_cheb(p2[0], p2[1], x0)
    p3 = spmm(x2, col, row, w)
    out = project(x0, x1, x2, p3[0], p3[1], weight, bias.reshape(1, Fout))
    return out.T[None, :, :, None, None, None]
